# baseline (speedup 1.0000x reference)
import dataclasses
import functools

import jax
import jax.numpy as jnp
from jax import lax
from jax.experimental import pallas as pl
from jax.experimental.pallas import tpu as pltpu
from jax.experimental.pallas import tpu_sc as plsc

NT = 32
TPS = 16
EC = 128
QC = 80


def _sc_mesh():
    return plsc.VectorSubcoreMesh(core_axis_name="c", subcore_axis_name="s")


def _make_sc_agg(N, D, CPT, with_cnt):
    RC = 80
    nrc = N // RC
    NP = N + 8

    out_type = [jax.ShapeDtypeStruct((2, N, D), jnp.float32)]
    scratch = [
        pltpu.VMEM((CPT, EC), jnp.int32),
        pltpu.VMEM((2, EC), jnp.int32),
        pltpu.VMEM((EC, D), jnp.float32),
        pltpu.VMEM_SHARED((NP, D), jnp.float32),
        pltpu.SemaphoreType.DMA,
        pltpu.SemaphoreType.DMA,
        pltpu.SemaphoreType.DMA,
    ]
    if with_cnt:
        out_type.append(jax.ShapeDtypeStruct((NT * N,), jnp.float32))
        scratch.append(pltpu.VMEM((NP,), jnp.float32))

    @functools.partial(
        pl.kernel, out_type=tuple(out_type), mesh=_sc_mesh(),
        scratch_types=scratch,
        compiler_params=dataclasses.replace(pltpu.CompilerParams(),
                                            needs_layout_passes=False))
    def sc_agg(h_hbm, src_hbm, dst_hbm, part_hbm, *rest):
        if with_cnt:
            cnt_hbm, dslab, sv, rows, accum, semg, semi0, semi1, hist = rest
        else:
            dslab, sv, rows, accum, semg, semi0, semi1 = rest
        c = lax.axis_index("c")
        s = lax.axis_index("s")
        wid = c * TPS + s
        tbase = wid * CPT

        @pl.loop(0, RC)
        def _(i):
            for j in range(0, D, 16):
                rows[i, pl.ds(j, 16)] = jnp.zeros((16,), jnp.float32)

        if with_cnt:
            @pl.loop(0, NP, step=16)
            def _(i):
                hist[pl.ds(i, 16)] = jnp.zeros((16,), jnp.float32)

        @pl.loop(s, nrc, step=TPS)
        def _(k):
            pltpu.sync_copy(rows.at[pl.ds(0, RC)], accum.at[pl.ds(k * RC, RC)])
        plsc.subcore_barrier()

        pltpu.sync_copy(dst_hbm.at[pl.ds(tbase, CPT)], dslab)
        pltpu.make_async_copy(src_hbm.at[pl.ds(tbase, 1)],
                              sv.at[pl.ds(0, 1)], semi0).start()
        pltpu.make_async_copy(src_hbm.at[pl.ds(tbase + 1, 1)],
                              sv.at[pl.ds(1, 1)], semi1).start()

        ones = jnp.ones((16,), jnp.float32)

        @pl.loop(0, CPT, step=2)
        def _(k):
            for b in range(2):
                j = k + b
                semi = semi0 if b == 0 else semi1
                pltpu.make_async_copy(src_hbm.at[pl.ds(tbase + j, 1)],
                                      sv.at[pl.ds(b, 1)], semi).wait()
                gat = pltpu.async_copy(h_hbm.at[sv.at[b]], rows, semg)
                if with_cnt:
                    for g in range(EC // 16):
                        plsc.addupdate_scatter(
                            hist, [dslab[j, pl.ds(g * 16, 16)]], ones)
                gat.wait()

                @pl.when(j + 2 < CPT)
                def _():
                    pltpu.make_async_copy(src_hbm.at[pl.ds(tbase + j + 2, 1)],
                                          sv.at[pl.ds(b, 1)], semi).start()
                pltpu.sync_copy(rows, accum.at[dslab.at[j]], add=True)

        plsc.subcore_barrier()

        @pl.loop(s, nrc, step=TPS)
        def _(k):
            off = k * RC
            pltpu.sync_copy(accum.at[pl.ds(off, RC)],
                            part_hbm.at[c, pl.ds(off, RC)])
        if with_cnt:
            pltpu.sync_copy(hist.at[pl.ds(0, N)],
                            cnt_hbm.at[pl.ds(wid * N, N)])

    return sc_agg


def _make_sc_pred(N, D, Q):
    nchunk = Q // QC

    @functools.partial(
        pl.kernel,
        out_type=jax.ShapeDtypeStruct((Q, 16), jnp.float32),
        mesh=_sc_mesh(),
        scratch_types=[
            pltpu.VMEM((QC,), jnp.int32),
            pltpu.VMEM((QC,), jnp.int32),
            pltpu.VMEM((QC, D), jnp.float32),
            pltpu.VMEM((QC, D), jnp.float32),
            pltpu.VMEM((QC, 16), jnp.float32),
            pltpu.SemaphoreType.DMA,
        ])
    def sc_pred(h_hbm, qa_hbm, qb_hbm, pred_hbm, ia, ib, ra, rb, dots, sem):
        c = lax.axis_index("c")
        s = lax.axis_index("s")
        wid = c * TPS + s

        @pl.loop(wid, nchunk, step=NT)
        def _(j):
            base = j * QC
            pltpu.sync_copy(qa_hbm.at[pl.ds(base, QC)], ia)
            pltpu.sync_copy(qb_hbm.at[pl.ds(base, QC)], ib)
            pltpu.async_copy(h_hbm.at[ia], ra, sem).wait()
            pltpu.async_copy(h_hbm.at[ib], rb, sem).wait()

            @pl.loop(0, QC)
            def _(r):
                acc = ra[r, pl.ds(0, 16)] * rb[r, pl.ds(0, 16)]
                for k in range(1, D // 16):
                    acc = acc + ra[r, pl.ds(16 * k, 16)] * rb[r, pl.ds(16 * k, 16)]
                dots[r, :] = acc

            pltpu.sync_copy(dots, pred_hbm.at[pl.ds(base, QC)])

    return sc_pred


def _make_tc_rowsum(Q):
    def body(pp_ref, o_ref):
        o_ref[...] = jnp.sum(pp_ref[...], axis=1)

    return pl.pallas_call(body,
                          out_shape=jax.ShapeDtypeStruct((Q,), jnp.float32))


def _make_tc_layer(N, D, first, leaky):
    def body(h_ref, p_ref, ci_ref, wl_ref, bl_ref, wr_ref, g_ref, b_ref,
             o_ref, *inv_out):
        if first:
            cnt = jnp.sum(ci_ref[...], axis=1, keepdims=True)
            inv = 1.0 / jnp.maximum(cnt, 1.0)
            inv_out[0][...] = inv
        else:
            inv = ci_ref[...]
        agg = (p_ref[0] + p_ref[1]) * inv
        out = jnp.dot(agg, wl_ref[...], preferred_element_type=jnp.float32)
        out = out + jnp.dot(h_ref[...], wr_ref[...],
                            preferred_element_type=jnp.float32)
        out = out + bl_ref[...]
        nrm = jnp.sqrt(jnp.sum(out * out, axis=1, keepdims=True))
        out = out / jnp.maximum(nrm, 1e-12)
        m = jnp.mean(out, axis=0, keepdims=True)
        d = out - m
        v = jnp.mean(d * d, axis=0, keepdims=True)
        out = d * (g_ref[...] / jnp.sqrt(v + 1e-5)) + b_ref[...]
        if leaky:
            out = jnp.where(out > 0.0, out, 0.01 * out)
        o_ref[...] = out

    out_shape = [jax.ShapeDtypeStruct((N, D), jnp.float32)]
    if first:
        out_shape.append(jax.ShapeDtypeStruct((N, 1), jnp.float32))
    return pl.pallas_call(body, out_shape=out_shape)


def kernel(x, edge_index, edge_label_index, Wl0, bl0, Wr0, gamma0, beta0,
           Wl1, bl1, Wr1, gamma1, beta1, Wl2, bl2, Wr2, gamma2, beta2):
    N, D = x.shape
    E = edge_index.shape[1]
    Q = edge_label_index.shape[1]
    src, dst = edge_index[0], edge_index[1]
    qa, qb = edge_label_index[0], edge_label_index[1]

    CPT = -(-E // (NT * EC))
    CPT += CPT % 2
    EP = NT * CPT * EC
    src2d = jnp.concatenate(
        [src, jnp.zeros((EP - E,), jnp.int32)]).reshape(NT * CPT, EC)
    dst2d = jnp.concatenate(
        [dst, jnp.full((EP - E,), N, jnp.int32)]).reshape(NT * CPT, EC)

    sc_agg0 = _make_sc_agg(N, D, CPT, with_cnt=True)
    sc_agg = _make_sc_agg(N, D, CPT, with_cnt=False)
    sc_pred = _make_sc_pred(N, D, Q)

    params = [(Wl0, bl0, Wr0, gamma0, beta0), (Wl1, bl1, Wr1, gamma1, beta1),
              (Wl2, bl2, Wr2, gamma2, beta2)]

    h = x
    inv = None
    for i, (Wl, bl, Wr, g, b) in enumerate(params):
        if i == 0:
            parts, cntflat = sc_agg0(h, src2d, dst2d)
            ci = cntflat.reshape(NT, N).T
        else:
            (parts,) = sc_agg(h, src2d, dst2d)
            ci = inv
        tc = _make_tc_layer(N, D, first=(i == 0), leaky=(i < 2))
        outs = tc(h, parts, ci, Wl, bl.reshape(1, D), Wr,
                  g.reshape(1, D), b.reshape(1, D))
        if i == 0:
            h, inv = outs
        else:
            h = outs[0]

    pp = sc_pred(h, qa, qb)
    pred = _make_tc_rowsum(Q)(pp)
    return (pred, h)
